# baseline (speedup 1.0000x reference)
import functools

import jax
import jax.numpy as jnp
from jax import lax
from jax.experimental import pallas as pl
from jax.experimental.pallas import tpu as pltpu
from jax.experimental.pallas import tpu_sc as plsc

D = 64
NBUF = 2
BU = 256
PITCH = 129


@functools.cache
def _make_sc_gather(T: int, B0: int):
    n_workers = 32
    bw = B0 // n_workers
    upt = bw // BU
    n_units = T * upt
    n_rounds = n_units // NBUF
    mesh = plsc.VectorSubcoreMesh(core_axis_name="c", subcore_axis_name="s")

    @functools.partial(
        pl.kernel,
        mesh=mesh,
        compiler_params=pltpu.CompilerParams(needs_layout_passes=False),
        out_type=jax.ShapeDtypeStruct((T, D, B0), jnp.float32),
        scratch_types=[
            pltpu.VMEM((NBUF, BU), jnp.int32),
            pltpu.VMEM((NBUF, 2, 128), jnp.int32),
            pltpu.VMEM((NBUF, BU), jnp.int32),
            pltpu.VMEM((NBUF, BU, 128), jnp.float32),
            pltpu.VMEM((128 * PITCH,), jnp.float32),
            pltpu.VMEM((NBUF, D, BU), jnp.float32),
            pltpu.SemaphoreType.DMA((NBUF,)),
            pltpu.SemaphoreType.DMA((NBUF,)),
            pltpu.SemaphoreType.DMA((NBUF,)),
        ],
    )
    def k(wp_hbm, xt_hbm, out_hbm, idx_v, idxp_v, par_v, gbuf, ibuf, tbuf,
          isem, gsem, osem):
        wid = lax.axis_index("s") * 2 + lax.axis_index("c")
        col0 = wid * bw

        iota = lax.iota(jnp.int32, 16)
        scat_base = [(iota + kk * 16) * PITCH for kk in range(8)]

        def unit_tb(u):
            return u // upt, col0 + (u % upt) * BU

        def idx_load(u, slot):
            t, b0 = unit_tb(u)
            return pltpu.make_async_copy(
                xt_hbm.at[t, pl.ds(b0, BU)], idx_v.at[slot], isem.at[slot]
            )

        def prep_idx(slot):
            for g in range(BU // 16):
                iv = idx_v[slot, pl.ds(g * 16, 16)]
                idxp_v[slot, g // 8, pl.ds((g % 8) * 16, 16)] = (
                    lax.shift_right_logical(iv, 1)
                )
                par_v[slot, pl.ds(g * 16, 16)] = lax.bitwise_and(iv, 1)

        def gather_halves(slot):
            return [
                pltpu.make_async_copy(
                    wp_hbm.at[idxp_v.at[slot, h]],
                    gbuf.at[slot, pl.ds(h * 128, 128)],
                    gsem.at[slot],
                )
                for h in range(2)
            ]

        def write(u, slot):
            t, b0 = unit_tb(u)
            return pltpu.make_async_copy(
                tbuf.at[slot], out_hbm.at[t, :, pl.ds(b0, BU)], osem.at[slot]
            )

        def transpose(slot):
            for h in range(2):
                hb = h * 128

                def abody(i, carry):
                    for j in range(2):
                        b = i * 2 + j
                        for kk in range(8):
                            v = gbuf[slot, hb + b, pl.ds(kk * 16, 16)]
                            plsc.store_scatter(ibuf, [scat_base[kk] + b], v)
                    return carry

                lax.fori_loop(0, 64, abody, 0)

                bases = []
                for bg in range(8):
                    pv = par_v[slot, pl.ds(hb + bg * 16, 16)]
                    bases.append(lax.shift_left(pv, 6) * PITCH + bg * 16 + iota)

                def bbody(d, carry):
                    off = d * PITCH
                    for bg in range(8):
                        v = plsc.load_gather(ibuf, [bases[bg] + off])
                        tbuf[slot, d, pl.ds(hb + bg * 16, 16)] = v
                    return carry

                lax.fori_loop(0, D, bbody, 0)

        for s in range(NBUF):
            idx_load(s, s).start()
            idx_load(s, s).wait()
            prep_idx(s)
            for c in gather_halves(s):
                c.start()

        def round_body(r, carry):
            for slot in range(NBUF):
                u = r * NBUF + slot
                for c in gather_halves(slot):
                    c.wait()
                nxt = u + NBUF

                @pl.when(nxt < n_units)
                def _():
                    idx_load(nxt, slot).start()

                @pl.when(u >= NBUF)
                def _():
                    write(u - NBUF, slot).wait()

                transpose(slot)
                write(u, slot).start()

                @pl.when(nxt < n_units)
                def _():
                    idx_load(nxt, slot).wait()
                    prep_idx(slot)
                    for c in gather_halves(slot):
                        c.start()

            return carry

        lax.fori_loop(0, n_rounds, round_body, 0)

        for s in range(NBUF):
            write(n_units - NBUF + s, s).wait()

    return k


def kernel(x, W):
    B0, T = x.shape
    wp = W.reshape(W.shape[0] // 2, 2 * W.shape[1])
    xt = x.T.astype(jnp.int32)
    k = _make_sc_gather(T, B0)
    out = k(wp, xt)
    return jnp.transpose(out, (2, 0, 1))


# speedup vs baseline: 1.3766x; 1.3766x over previous
import functools

import jax
import jax.numpy as jnp
from jax import lax
from jax.experimental import pallas as pl
from jax.experimental.pallas import tpu as pltpu
from jax.experimental.pallas import tpu_sc as plsc

EMBEDDING_DIM = 64


@functools.cache
def _make_flatten(T: int, B0: int):
    n_workers = 32
    bw = B0 // n_workers
    mesh = plsc.VectorSubcoreMesh(core_axis_name="c", subcore_axis_name="s")

    @functools.partial(
        pl.kernel,
        mesh=mesh,
        compiler_params=pltpu.CompilerParams(needs_layout_passes=False),
        out_type=jax.ShapeDtypeStruct((T * B0,), jnp.int32),
        scratch_types=[
            pltpu.VMEM((T, bw), jnp.int32),
            pltpu.VMEM((T * bw,), jnp.int32),
        ],
    )
    def k(xt_hbm, flat_hbm, inb, outb):
        wid = lax.axis_index("s") * 2 + lax.axis_index("c")
        col0 = wid * bw
        pltpu.sync_copy(xt_hbm.at[:, pl.ds(col0, bw)], inb)

        iota = lax.iota(jnp.int32, 16)

        def tbody(t, carry):
            for g in range(bw // 16):
                v = inb[t, pl.ds(g * 16, 16)]
                addr = (iota + g * 16) * T + t
                plsc.store_scatter(outb, [addr], v)
            return carry

        lax.fori_loop(0, T, tbody, 0)
        pltpu.sync_copy(outb, flat_hbm.at[pl.ds(col0 * T, bw * T)])

    return k


@functools.cache
def _make_sc_gather(B: int, D: int, n_workers: int, chunk: int, nbuf: int):
    b_per_w = B // n_workers
    n_chunks = b_per_w // chunk
    n_rounds = (n_chunks + nbuf - 1) // nbuf
    mesh = plsc.VectorSubcoreMesh(core_axis_name="c", subcore_axis_name="s")

    @functools.partial(
        pl.kernel,
        mesh=mesh,
        compiler_params=pltpu.CompilerParams(use_tc_tiling_on_sc=False),
        out_type=jax.ShapeDtypeStruct((B, D), jnp.float32),
        scratch_types=[
            pltpu.VMEM((b_per_w,), jnp.int32),
            pltpu.VMEM((nbuf, chunk, D), jnp.float32),
            pltpu.SemaphoreType.DMA((nbuf,)),
            pltpu.SemaphoreType.DMA((nbuf,)),
        ],
    )
    def k(table_hbm, idx_hbm, out_hbm, idx_v, rows_v, gsem, osem):
        wid = lax.axis_index("s") * 2 + lax.axis_index("c")
        base0 = wid * b_per_w
        pltpu.sync_copy(idx_hbm.at[pl.ds(base0, b_per_w)], idx_v)

        def gather(i, b):
            off = pl.multiple_of(i * chunk, chunk)
            return pltpu.make_async_copy(
                table_hbm.at[idx_v.at[pl.ds(off, chunk)]], rows_v.at[b],
                gsem.at[b]
            )

        def write(i, b):
            off = pl.multiple_of(base0 + i * chunk, chunk)
            return pltpu.make_async_copy(
                rows_v.at[b], out_hbm.at[pl.ds(off, chunk)], osem.at[b]
            )

        for b in range(nbuf):
            gather(b, b).start()

        def round_body(r, carry):
            for b in range(nbuf):
                i = r * nbuf + b

                @pl.when(i < n_chunks)
                def _():
                    gather(i, b).wait()
                    write(i, b).start()
                    nxt = i + nbuf

                    @pl.when(nxt < n_chunks)
                    def _():
                        write(i, b).wait()
                        gather(nxt, b).start()

            return carry

        lax.fori_loop(0, n_rounds, round_body, 0)

        for b in range(nbuf):
            last_i = ((n_chunks - 1 - b) // nbuf) * nbuf + b
            write(last_i, b).wait()

    return k


def kernel(x, W):
    B0, T = x.shape
    B = B0 * T
    xt = x.T.astype(jnp.int32)
    flat_idx = _make_flatten(T, B0)(xt)
    gather = _make_sc_gather(B, EMBEDDING_DIM, 32, 512, 3)
    out = gather(W, flat_idx)
    return out.reshape(B0, T, EMBEDDING_DIM)
